# baseline (speedup 1.0000x reference)
import functools

import jax
import jax.numpy as jnp
from jax import lax
from jax.experimental import pallas as pl
from jax.experimental.pallas import tpu as pltpu
from jax.experimental.pallas import tpu_sc as plsc

_S_CHUNK = 512
_B_BLK = 4


def _add_pos_kernel(x_ref, pos_ref, scale_ref, out_ref):
    out_ref[...] = x_ref[...] + pos_ref[...] * scale_ref[0]


def _tc_kernel(x, pos, scale):
    batch, seq_len, dim = x.shape
    num_chunks = seq_len // _S_CHUNK
    return pl.pallas_call(
        _add_pos_kernel,
        grid=(num_chunks, batch // _B_BLK),
        in_specs=[
            pl.BlockSpec((_B_BLK, _S_CHUNK, dim), lambda i, j: (j, i, 0)),
            pl.BlockSpec((_S_CHUNK, dim), lambda i, j: (i, 0)),
            pl.BlockSpec(memory_space=pltpu.SMEM),
        ],
        out_specs=pl.BlockSpec((_B_BLK, _S_CHUNK, dim), lambda i, j: (j, i, 0)),
        out_shape=jax.ShapeDtypeStruct(x.shape, x.dtype),
    )(x, pos, scale)


_LANES = 16
_R = 16
_SEQ_BLOCKS = 8
_ROWS_PER_W = 1024
_NCHUNK = _ROWS_PER_W // _R


def _sc_body(x_hbm, pos_hbm, scale_hbm, out_hbm, xb, pb, ob, sb,
             in_sem0, in_sem1, out_sem0, out_sem1):
    dim = xb.shape[2]
    in_sems = (in_sem0, in_sem1)
    out_sems = (out_sem0, out_sem1)
    c = lax.axis_index("c")
    s = lax.axis_index("s")
    wid = c * 16 + s
    b = wid // _SEQ_BLOCKS
    s0 = (wid % _SEQ_BLOCKS) * _ROWS_PER_W

    pltpu.sync_copy(scale_hbm, sb)
    vscale = sb[...]

    def x_src(k):
        return x_hbm.at[b, pl.ds(s0 + k * _R, _R), :]

    def pos_src(k):
        return pos_hbm.at[pl.ds(s0 + k * _R, _R), :]

    def out_dst(k):
        return out_hbm.at[b, pl.ds(s0 + k * _R, _R), :]

    def start_in(k, slot):
        pltpu.async_copy(x_src(k), xb.at[slot], in_sems[slot])
        pltpu.async_copy(pos_src(k), pb.at[slot], in_sems[slot])

    start_in(0, 0)
    start_in(1, 1)

    def pair_body(g, carry):
        for slot in range(2):
            k = 2 * g + slot
            pltpu.make_async_copy(x_src(k), xb.at[slot], in_sems[slot]).wait()
            pltpu.make_async_copy(pos_src(k), pb.at[slot], in_sems[slot]).wait()

            @pl.when(g >= 1)
            def _():
                pltpu.make_async_copy(
                    ob.at[slot], out_dst(k - 2), out_sems[slot]).wait()

            def row_body(i, carry2):
                for j in range(dim // _LANES):
                    col = pl.ds(j * _LANES, _LANES)
                    ob[slot, i, col] = xb[slot, i, col] + pb[slot, i, col] * vscale
                return carry2

            lax.fori_loop(0, _R, row_body, 0)

            pltpu.async_copy(ob.at[slot], out_dst(k), out_sems[slot])

            @pl.when(k + 2 < _NCHUNK)
            def _():
                start_in(k + 2, slot)
        return carry

    lax.fori_loop(0, _NCHUNK // 2, pair_body, 0)

    pltpu.make_async_copy(ob.at[0], out_dst(_NCHUNK - 2), out_sem0).wait()
    pltpu.make_async_copy(ob.at[1], out_dst(_NCHUNK - 1), out_sem1).wait()


def _sc_kernel(x, pos, scale):
    batch, seq_len, dim = x.shape
    scale_vec = jnp.broadcast_to(scale, (_LANES,))
    mesh = plsc.VectorSubcoreMesh(core_axis_name="c", subcore_axis_name="s")
    run = functools.partial(
        pl.kernel,
        mesh=mesh,
        out_type=jax.ShapeDtypeStruct(x.shape, x.dtype),
        scratch_types=[
            pltpu.VMEM((2, _R, dim), jnp.float32),
            pltpu.VMEM((2, _R, dim), jnp.float32),
            pltpu.VMEM((2, _R, dim), jnp.float32),
            pltpu.VMEM((_LANES,), jnp.float32),
            pltpu.SemaphoreType.DMA,
            pltpu.SemaphoreType.DMA,
            pltpu.SemaphoreType.DMA,
            pltpu.SemaphoreType.DMA,
        ],
    )(_sc_body)
    return run(x, pos, scale_vec)


def kernel(x, pos_embedding, scale):
    seq_len = x.shape[1]
    pos = pos_embedding[:seq_len]
    return _tc_kernel(x, pos, scale)


# speedup vs baseline: 1.0109x; 1.0109x over previous
import functools

import jax
import jax.numpy as jnp
from jax import lax
from jax.experimental import pallas as pl
from jax.experimental.pallas import tpu as pltpu
from jax.experimental.pallas import tpu_sc as plsc

_S_CHUNK = 2048
_B_BLK = 1


def _add_pos_kernel(x_ref, pos_ref, scale_ref, out_ref):
    out_ref[...] = x_ref[...] + pos_ref[...] * scale_ref[0]


def _tc_kernel(x, pos, scale):
    batch, seq_len, dim = x.shape
    num_chunks = seq_len // _S_CHUNK
    return pl.pallas_call(
        _add_pos_kernel,
        grid=(num_chunks, batch // _B_BLK),
        in_specs=[
            pl.BlockSpec((_B_BLK, _S_CHUNK, dim), lambda i, j: (j, i, 0)),
            pl.BlockSpec((_S_CHUNK, dim), lambda i, j: (i, 0)),
            pl.BlockSpec(memory_space=pltpu.SMEM),
        ],
        out_specs=pl.BlockSpec((_B_BLK, _S_CHUNK, dim), lambda i, j: (j, i, 0)),
        out_shape=jax.ShapeDtypeStruct(x.shape, x.dtype),
    )(x, pos, scale)


_LANES = 16
_R = 16
_SEQ_BLOCKS = 8
_ROWS_PER_W = 1024
_NCHUNK = _ROWS_PER_W // _R


def _sc_body(x_hbm, pos_hbm, scale_hbm, out_hbm, xb, pb, ob, sb,
             in_sem0, in_sem1, out_sem0, out_sem1):
    dim = xb.shape[2]
    in_sems = (in_sem0, in_sem1)
    out_sems = (out_sem0, out_sem1)
    c = lax.axis_index("c")
    s = lax.axis_index("s")
    wid = c * 16 + s
    b = wid // _SEQ_BLOCKS
    s0 = (wid % _SEQ_BLOCKS) * _ROWS_PER_W

    pltpu.sync_copy(scale_hbm, sb)
    vscale = sb[...]

    def x_src(k):
        return x_hbm.at[b, pl.ds(s0 + k * _R, _R), :]

    def pos_src(k):
        return pos_hbm.at[pl.ds(s0 + k * _R, _R), :]

    def out_dst(k):
        return out_hbm.at[b, pl.ds(s0 + k * _R, _R), :]

    def start_in(k, slot):
        pltpu.async_copy(x_src(k), xb.at[slot], in_sems[slot])
        pltpu.async_copy(pos_src(k), pb.at[slot], in_sems[slot])

    start_in(0, 0)
    start_in(1, 1)

    def pair_body(g, carry):
        for slot in range(2):
            k = 2 * g + slot
            pltpu.make_async_copy(x_src(k), xb.at[slot], in_sems[slot]).wait()
            pltpu.make_async_copy(pos_src(k), pb.at[slot], in_sems[slot]).wait()

            @pl.when(g >= 1)
            def _():
                pltpu.make_async_copy(
                    ob.at[slot], out_dst(k - 2), out_sems[slot]).wait()

            def row_body(i, carry2):
                for j in range(dim // _LANES):
                    col = pl.ds(j * _LANES, _LANES)
                    ob[slot, i, col] = xb[slot, i, col] + pb[slot, i, col] * vscale
                return carry2

            lax.fori_loop(0, _R, row_body, 0)

            pltpu.async_copy(ob.at[slot], out_dst(k), out_sems[slot])

            @pl.when(k + 2 < _NCHUNK)
            def _():
                start_in(k + 2, slot)
        return carry

    lax.fori_loop(0, _NCHUNK // 2, pair_body, 0)

    pltpu.make_async_copy(ob.at[0], out_dst(_NCHUNK - 2), out_sem0).wait()
    pltpu.make_async_copy(ob.at[1], out_dst(_NCHUNK - 1), out_sem1).wait()


def _sc_kernel(x, pos, scale):
    batch, seq_len, dim = x.shape
    scale_vec = jnp.broadcast_to(scale, (_LANES,))
    mesh = plsc.VectorSubcoreMesh(core_axis_name="c", subcore_axis_name="s")
    run = functools.partial(
        pl.kernel,
        mesh=mesh,
        out_type=jax.ShapeDtypeStruct(x.shape, x.dtype),
        scratch_types=[
            pltpu.VMEM((2, _R, dim), jnp.float32),
            pltpu.VMEM((2, _R, dim), jnp.float32),
            pltpu.VMEM((2, _R, dim), jnp.float32),
            pltpu.VMEM((_LANES,), jnp.float32),
            pltpu.SemaphoreType.DMA,
            pltpu.SemaphoreType.DMA,
            pltpu.SemaphoreType.DMA,
            pltpu.SemaphoreType.DMA,
        ],
    )(_sc_body)
    return run(x, pos, scale_vec)


def kernel(x, pos_embedding, scale):
    seq_len = x.shape[1]
    pos = pos_embedding[:seq_len]
    return _tc_kernel(x, pos, scale)
